# baseline (speedup 1.0000x reference)
import functools

import jax
import jax.numpy as jnp
from jax import lax
from jax.experimental import pallas as pl
from jax.experimental.pallas import tpu as pltpu
from jax.experimental.pallas import tpu_sc as plsc

N, E, T = 10000, 160000, 320000
ATOM, BOND, ANGLE = 128, 128, 16
C2 = 2 * BOND

_SC_INFO = plsc.get_sparse_core_info()
_NC = _SC_INFO.num_cores
_NS = _SC_INFO.num_subcores
NW = _NC * _NS


_GB = 80
_CHUNK = T // NW
_NBLK = _CHUNK // _GB


_CW = C2 // 2


def _bf2(v):
    return plsc.bitcast(v, jnp.bfloat16)


def _gather_sum_body(pa_hbm, pj_hbm, pi_hbm, pk_hbm, j_hbm, i_hbm, k_hbm,
                     x_hbm, jb, ib, kb, xa, gj, gi, gk,
                     sem_a, sem_j, sem_i, sem_k):
    wid = lax.axis_index("s") * _NC + lax.axis_index("c")

    def blk_body(b, carry):
        base = wid * _CHUNK + b * _GB
        pltpu.sync_copy(j_hbm.at[pl.ds(base, _GB)], jb)
        pltpu.sync_copy(i_hbm.at[pl.ds(base, _GB)], ib)
        pltpu.sync_copy(k_hbm.at[pl.ds(base, _GB)], kb)
        ca = pltpu.async_copy(pa_hbm.at[pl.ds(base, _GB)], xa, sem_a)
        cj = pltpu.async_copy(pj_hbm.at[jb], gj, sem_j)
        ci = pltpu.async_copy(pi_hbm.at[ib], gi, sem_i)
        ck = pltpu.async_copy(pk_hbm.at[kb], gk, sem_k)
        ca.wait()
        cj.wait()
        ci.wait()
        ck.wait()

        def row_body(r, c2):
            for c in range(_CW // 16):
                sl = pl.ds(c * 16, 16)
                s = (_bf2(xa[r, sl]) + _bf2(gj[r, sl])) +                     (_bf2(gi[r, sl]) + _bf2(gk[r, sl]))
                xa[r, sl] = plsc.bitcast(s, jnp.int32)
            return c2

        lax.fori_loop(0, _GB, row_body, 0)
        pltpu.sync_copy(xa, x_hbm.at[pl.ds(base, _GB)])
        return carry

    lax.fori_loop(0, _NBLK, blk_body, 0)


def _gather_sum(Pa, Pj, Pi, Pk, j_idx, i_idx, k_idx):
    mesh = plsc.VectorSubcoreMesh(core_axis_name="c", subcore_axis_name="s")
    f = functools.partial(
        pl.kernel,
        mesh=mesh,
        compiler_params=pltpu.CompilerParams(needs_layout_passes=False),
        out_type=jax.ShapeDtypeStruct((T, _CW), jnp.int32),
        scratch_types=[
            pltpu.VMEM((_GB,), jnp.int32),
            pltpu.VMEM((_GB,), jnp.int32),
            pltpu.VMEM((_GB,), jnp.int32),
            pltpu.VMEM((_GB, _CW), jnp.int32),
            pltpu.VMEM((_GB, _CW), jnp.int32),
            pltpu.VMEM((_GB, _CW), jnp.int32),
            pltpu.VMEM((_GB, _CW), jnp.int32),
            pltpu.SemaphoreType.DMA,
            pltpu.SemaphoreType.DMA,
            pltpu.SemaphoreType.DMA,
            pltpu.SemaphoreType.DMA,
        ],
    )(_gather_sum_body)
    return f(Pa, Pj, Pi, Pk, j_idx, i_idx, k_idx)


NBKT = E // 256
_NBC = 640
_CAP = 14528
_CHG = _CHUNK // 16

_UW = BOND // 2


def _bucketize_body(k_hbm, perm_hbm, kperm_hbm, cnt_hbm, loff_hbm,
                    kb_v, perm_v, kp_v, io_v, cnt_s, sem0):
    wid = lax.axis_index("s") * _NC + lax.axis_index("c")
    base = wid * _CHUNK
    lanes = lax.iota(jnp.int32, 16)
    mask0 = lanes == 0
    zero16 = jnp.zeros((16,), jnp.int32)

    def z_body(g, c):
        perm_v[pl.ds(pl.multiple_of(g * 16, 8), 16)] = zero16
        return c
    lax.fori_loop(0, _CAP // 16, z_body, 0)

    def zc_body(b, c):
        cnt_s[b] = 0
        return c
    lax.fori_loop(0, _NBC, zc_body, 0)

    def h_outer(blk5, c5):
        pltpu.async_copy(k_hbm.at[pl.ds(base + blk5 * 2000, 2000)], kb_v,
                         sem0).wait()

        def h_body(g, c):
            kv = kb_v[pl.ds(pl.multiple_of(g * 16, 8), 16)]
            for lane in range(16):
                b = kv[lane] >> 8
                cnt_s[b] = cnt_s[b] + 1
            return c
        lax.fori_loop(0, 125, h_body, 0)
        return c5
    lax.fori_loop(0, 5, h_outer, 0)

    def x1_body(b, c):
        plsc.store_scatter(io_v, [jnp.full((16,), b, jnp.int32)],
                           jnp.full((16,), cnt_s[b], jnp.int32), mask=mask0)
        return c
    lax.fori_loop(0, _NBC, x1_body, 0)
    pltpu.sync_copy(io_v, cnt_hbm.at[pl.ds(wid * _NBC, _NBC)])

    def p_body(b, run):
        c = cnt_s[b]
        cnt_s[b] = run
        return run + ((c + 7) & -8)
    lax.fori_loop(0, _NBC, p_body, jnp.int32(0))

    def x2_body(b, c):
        plsc.store_scatter(io_v, [jnp.full((16,), b, jnp.int32)],
                           jnp.full((16,), cnt_s[b], jnp.int32), mask=mask0)
        return c
    lax.fori_loop(0, _NBC, x2_body, 0)
    pltpu.sync_copy(io_v, loff_hbm.at[pl.ds(wid * _NBC, _NBC)])

    def s_outer(blk5, c5):
        pltpu.async_copy(k_hbm.at[pl.ds(base + blk5 * 2000, 2000)], kb_v,
                         sem0).wait()

        def s_body(g, c):
            g16 = pl.multiple_of(g * 16, 8)
            kv = kb_v[pl.ds(g16, 16)]
            for lane in range(16):
                k = kv[lane]
                b = k >> 8
                o = cnt_s[b]
                cnt_s[b] = o + 1
                t = base + blk5 * 2000 + g16 + lane
                oi = jnp.full((16,), o, jnp.int32)
                plsc.store_scatter(perm_v, [oi], jnp.full((16,), t, jnp.int32),
                                   mask=mask0)
                plsc.store_scatter(kp_v, [oi], jnp.full((16,), k, jnp.int32),
                                   mask=mask0)
            return c
        lax.fori_loop(0, 125, s_body, 0)
        return c5
    lax.fori_loop(0, 5, s_outer, 0)

    pltpu.sync_copy(perm_v, perm_hbm.at[pl.ds(wid * _CAP, _CAP)])
    pltpu.sync_copy(kp_v, kperm_hbm.at[pl.ds(wid * _CAP, _CAP)])


def _bucketize(k_idx):
    mesh = plsc.VectorSubcoreMesh(core_axis_name="c", subcore_axis_name="s")
    f = functools.partial(
        pl.kernel,
        mesh=mesh,
        compiler_params=pltpu.CompilerParams(needs_layout_passes=False),
        out_type=(
            jax.ShapeDtypeStruct((NW * _CAP,), jnp.int32),
            jax.ShapeDtypeStruct((NW * _CAP,), jnp.int32),
            jax.ShapeDtypeStruct((NW * _NBC,), jnp.int32),
            jax.ShapeDtypeStruct((NW * _NBC,), jnp.int32),
        ),
        scratch_types=[
            pltpu.VMEM((2000,), jnp.int32),
            pltpu.VMEM((_CAP,), jnp.int32),
            pltpu.VMEM((_CAP,), jnp.int32),
            pltpu.VMEM((_NBC,), jnp.int32),
            pltpu.SMEM((_NBC,), jnp.int32),
            pltpu.SemaphoreType.DMA,
        ],
    )(_bucketize_body)
    return f(k_idx)


def _accum_body(u_hbm, perm_hbm, kperm_hbm, cntT_hbm, loffT_hbm, seg_hbm,
                cnt_v, loff_v, acc, tb, kb2, us, sem0, sem1):
    wid = lax.axis_index("s") * _NC + lax.axis_index("c")
    zero16 = jnp.zeros((16,), jnp.float32)
    brow = wid * 20
    skew = brow & 4
    start8 = pl.multiple_of(brow - skew, 8)
    c0 = pltpu.async_copy(cntT_hbm.at[pl.ds(start8, 24)], cnt_v, sem0)
    c1 = pltpu.async_copy(loffT_hbm.at[pl.ds(start8, 24)], loff_v, sem1)
    c0.wait()
    c1.wait()
    nb = jnp.maximum(0, jnp.minimum(20, NBKT - brow))

    def bkt_body(lb0, cb):
        lb = lb0 + skew
        b = brow + lb0
        def za_body(r, c):
            for c4 in range(8):
                acc[r, pl.ds(c4 * 16, 16)] = zero16
            return c
        lax.fori_loop(0, 256, za_body, 0)

        def p_body(p, cp):
            n = cnt_v[lb, pl.ds(p, 16)][0]
            l = loff_v[lb, pl.ds(p, 16)][0]
            pbase = p * _CAP + l

            @pl.when(n > 0)
            def _():
                nchunks = (n + 63) >> 6

                def ch_body(ch, cc):
                    off = pl.multiple_of(pbase + ch * 64, 8)
                    pltpu.sync_copy(perm_hbm.at[pl.ds(off, 64)], tb)
                    pltpu.sync_copy(kperm_hbm.at[pl.ds(off, 64)], kb2)
                    pltpu.async_copy(u_hbm.at[tb], us, sem0).wait()
                    rem = n - ch * 64
                    for g in range(4):
                        @pl.when(g * 16 < rem)
                        def _():
                            kv = kb2[pl.ds(g * 16, 16)]
                            for lane in range(16):
                                @pl.when(g * 16 + lane < rem)
                                def _():
                                    r = kv[lane] & 255
                                    row = g * 16 + lane
                                    for c4 in range(8):
                                        sl = pl.ds(c4 * 16, 16)
                                        acc[r, sl] = acc[r, sl] + us[row, sl]
                    return cc
                lax.fori_loop(0, nchunks, ch_body, 0)
            return cp
        lax.fori_loop(0, NW, p_body, 0)
        pltpu.sync_copy(acc, seg_hbm.at[pl.ds(b * 256, 256)])
        return cb
    lax.fori_loop(0, nb, bkt_body, 0)


def _accumulate(u_packed, perm, kperm, cntT, loffT):
    mesh = plsc.VectorSubcoreMesh(core_axis_name="c", subcore_axis_name="s")
    f = functools.partial(
        pl.kernel,
        mesh=mesh,
        compiler_params=pltpu.CompilerParams(needs_layout_passes=False),
        out_type=jax.ShapeDtypeStruct((E, BOND), jnp.float32),
        scratch_types=[
            pltpu.VMEM((24, 48), jnp.int32),
            pltpu.VMEM((24, 48), jnp.int32),
            pltpu.VMEM((256, BOND), jnp.float32),
            pltpu.VMEM((64,), jnp.int32),
            pltpu.VMEM((64,), jnp.int32),
            pltpu.VMEM((64, BOND), jnp.float32),
            pltpu.SemaphoreType.DMA,
            pltpu.SemaphoreType.DMA,
        ],
    )(_accum_body)
    return f(u_packed, perm, kperm, cntT, loffT)


def _pack_pair(core_f32, gate_f32):
    cb = jax.lax.bitcast_convert_type(core_f32, jnp.uint32)
    gb = jax.lax.bitcast_convert_type(gate_f32, jnp.uint32)
    cb = (cb + jnp.uint32(0x8000)) >> 16
    gb = (gb + jnp.uint32(0x8000)) & jnp.uint32(0xFFFF0000)
    return jax.lax.bitcast_convert_type(cb | gb, jnp.int32)


def _unpack_pair(word_i32):
    w = jax.lax.bitcast_convert_type(word_i32, jnp.uint32)
    core = jax.lax.bitcast_convert_type(w << 16, jnp.float32)
    gate = jax.lax.bitcast_convert_type(w & jnp.uint32(0xFFFF0000), jnp.float32)
    return core, gate


def _mm_body(x_ref, w_ref, o_ref):
    y = jnp.dot(x_ref[...], w_ref[...], preferred_element_type=jnp.float32)
    o_ref[...] = _pack_pair(y[:, :BOND], y[:, BOND:])


def _rowmm_packed(x, w, blk):
    m, k = x.shape
    n = w.shape[1]
    return pl.pallas_call(
        _mm_body,
        grid=(m // blk,),
        in_specs=[pl.BlockSpec((blk, k), lambda i: (i, 0)),
                  pl.BlockSpec((k, n), lambda i: (0, 0))],
        out_specs=pl.BlockSpec((blk, n // 2), lambda i: (i, 0)),
        out_shape=jax.ShapeDtypeStruct((m, n // 2), jnp.int32),
    )(x, w)


def _stats_body(x_ref, o_ref):
    @pl.when(pl.program_id(0) == 0)
    def _():
        o_ref[...] = jnp.zeros_like(o_ref)
    core, gate = _unpack_pair(x_ref[...])
    o_ref[0, :] += jnp.sum(core, axis=0)
    o_ref[1, :] += jnp.sum(core * core, axis=0)
    o_ref[2, :] += jnp.sum(gate, axis=0)
    o_ref[3, :] += jnp.sum(gate * gate, axis=0)


def _stats(x, blk):
    m, n = x.shape
    return pl.pallas_call(
        _stats_body,
        grid=(m // blk,),
        in_specs=[pl.BlockSpec((blk, n), lambda i: (i, 0))],
        out_specs=pl.BlockSpec((8, n), lambda i: (0, 0)),
        out_shape=jax.ShapeDtypeStruct((8, n), jnp.float32),
    )(x)


def _act_body(x_ref, st_ref, p_ref, o_ref):
    core, gate = _unpack_pair(x_ref[...])
    mean_c = st_ref[0, :] / T
    var_c = st_ref[1, :] / T - mean_c * mean_c
    inv_c = jax.lax.rsqrt(var_c + 1e-5)
    mean_g = st_ref[2, :] / T
    var_g = st_ref[3, :] / T - mean_g * mean_g
    inv_g = jax.lax.rsqrt(var_g + 1e-5)
    core = (core - mean_c) * inv_c * p_ref[0, :] + p_ref[1, :]
    gate = (gate - mean_g) * inv_g * p_ref[2, :] + p_ref[3, :]
    core = core * jax.nn.sigmoid(core)
    gate = jax.nn.sigmoid(gate)
    o_ref[...] = core * gate


def _activate(x, stats, params, blk):
    m = x.shape[0]
    return pl.pallas_call(
        _act_body,
        grid=(m // blk,),
        in_specs=[pl.BlockSpec((blk, BOND), lambda i: (i, 0)),
                  pl.BlockSpec((8, BOND), lambda i: (0, 0)),
                  pl.BlockSpec((8, BOND), lambda i: (0, 0))],
        out_specs=pl.BlockSpec((blk, BOND), lambda i: (i, 0)),
        out_shape=jax.ShapeDtypeStruct((m, BOND), jnp.float32),
    )(x, stats, params)


def _final_body(seg_ref, edge_ref, w_ref, out_ref):
    out_ref[...] = jnp.dot(seg_ref[...], w_ref[...],
                           preferred_element_type=jnp.float32) + edge_ref[...]


def _final_matmul(segsum, edge_feat, W_out):
    BLK = 1600
    return pl.pallas_call(
        _final_body,
        grid=(E // BLK,),
        in_specs=[
            pl.BlockSpec((BLK, BOND), lambda i: (i, 0)),
            pl.BlockSpec((BLK, BOND), lambda i: (i, 0)),
            pl.BlockSpec((BOND, BOND), lambda i: (0, 0)),
        ],
        out_specs=pl.BlockSpec((BLK, BOND), lambda i: (i, 0)),
        out_shape=jax.ShapeDtypeStruct((E, BOND), jnp.float32),
    )(segsum, edge_feat, W_out)


def kernel(vertex_feat, edge_feat, angle_feat, edge_index, k_idx, j_idx, i_idx,
           W_core_src, W_core_dst, W_core_bond, W_core_angle,
           W_gate_src, W_gate_dst, W_gate_bond, W_gate_angle,
           bn_core_gamma, bn_core_beta, bn_gate_gamma, bn_gate_beta, W_out):
    k_idx = k_idx.astype(jnp.int32)
    j_idx = j_idx.astype(jnp.int32)
    i_idx = i_idx.astype(jnp.int32)

    Wj = jnp.concatenate([W_core_src, W_gate_src], axis=1)
    Wi = jnp.concatenate([W_core_dst, W_gate_dst], axis=1)
    Wk = jnp.concatenate([W_core_bond, W_gate_bond], axis=1)
    Wa = jnp.concatenate([W_core_angle, W_gate_angle], axis=1)
    Pj = _rowmm_packed(vertex_feat, Wj, 2000)
    Pi = _rowmm_packed(vertex_feat, Wi, 2000)
    Pk = _rowmm_packed(edge_feat, Wk, 4000)
    Pa = _rowmm_packed(angle_feat, Wa, 8000)

    x = _gather_sum(Pa, Pj, Pi, Pk, j_idx, i_idx, k_idx)

    stats = _stats(x, 8000)
    params = jnp.zeros((8, BOND), jnp.float32)
    params = params.at[0].set(bn_core_gamma).at[1].set(bn_core_beta)
    params = params.at[2].set(bn_gate_gamma).at[3].set(bn_gate_beta)
    u_packed = _activate(x, stats, params, 4000)

    perm, kperm, cnt, loff = _bucketize(k_idx)
    pad = jnp.zeros((_NBC, 48 - NW), jnp.int32)
    cntT = jnp.concatenate([cnt.reshape(NW, _NBC).T, pad], axis=1)
    loffT = jnp.concatenate([loff.reshape(NW, _NBC).T, pad], axis=1)
    segsum = _accumulate(u_packed, perm, kperm, cntT, loffT)

    return _final_matmul(segsum, edge_feat, W_out)


# speedup vs baseline: 3.2632x; 3.2632x over previous
import functools

import jax
import jax.numpy as jnp
from jax import lax
from jax.experimental import pallas as pl
from jax.experimental.pallas import tpu as pltpu
from jax.experimental.pallas import tpu_sc as plsc

N, E, T = 10000, 160000, 320000
ATOM, BOND, ANGLE = 128, 128, 16
C2 = 2 * BOND

_SC_INFO = plsc.get_sparse_core_info()
_NC = _SC_INFO.num_cores
_NS = _SC_INFO.num_subcores
NW = _NC * _NS


_GB = 40
_CHUNK = T // NW
_NBLK = _CHUNK // _GB


_CW = C2 // 2


def _bf2(v):
    return plsc.bitcast(v, jnp.bfloat16)


def _gather_sum_body(pa_hbm, pj_hbm, pi_hbm, pk_hbm, j_hbm, i_hbm, k_hbm,
                     x_hbm, jb, ib, kb, xa, gj, gi, gk,
                     sem_a, sem_j, sem_i, sem_k):
    wid = lax.axis_index("s") * _NC + lax.axis_index("c")

    def blk_body(b, carry):
        base = wid * _CHUNK + b * _GB
        pltpu.sync_copy(j_hbm.at[pl.ds(base, _GB)], jb)
        pltpu.sync_copy(i_hbm.at[pl.ds(base, _GB)], ib)
        pltpu.sync_copy(k_hbm.at[pl.ds(base, _GB)], kb)
        ca = pltpu.async_copy(pa_hbm.at[pl.ds(base, _GB)], xa, sem_a)
        cj = pltpu.async_copy(pj_hbm.at[jb], gj, sem_j)
        ci = pltpu.async_copy(pi_hbm.at[ib], gi, sem_i)
        ck = pltpu.async_copy(pk_hbm.at[kb], gk, sem_k)
        ca.wait()
        cj.wait()
        ci.wait()
        ck.wait()

        def row_body(r, c2):
            for c in range(_CW // 16):
                sl = pl.ds(c * 16, 16)
                s = (_bf2(xa[r, sl]) + _bf2(gj[r, sl])) +                     (_bf2(gi[r, sl]) + _bf2(gk[r, sl]))
                xa[r, sl] = plsc.bitcast(s, jnp.int32)
            return c2

        lax.fori_loop(0, _GB, row_body, 0)
        pltpu.sync_copy(xa, x_hbm.at[pl.ds(base, _GB)])
        return carry

    lax.fori_loop(0, _NBLK, blk_body, 0)


def _gather_sum(Pa, Pj, Pi, Pk, j_idx, i_idx, k_idx):
    mesh = plsc.VectorSubcoreMesh(core_axis_name="c", subcore_axis_name="s")
    f = functools.partial(
        pl.kernel,
        mesh=mesh,
        compiler_params=pltpu.CompilerParams(needs_layout_passes=False),
        out_type=jax.ShapeDtypeStruct((T, _CW), jnp.int32),
        scratch_types=[
            pltpu.VMEM((_GB,), jnp.int32),
            pltpu.VMEM((_GB,), jnp.int32),
            pltpu.VMEM((_GB,), jnp.int32),
            pltpu.VMEM((_GB, _CW), jnp.int32),
            pltpu.VMEM((_GB, _CW), jnp.int32),
            pltpu.VMEM((_GB, _CW), jnp.int32),
            pltpu.VMEM((_GB, _CW), jnp.int32),
            pltpu.SemaphoreType.DMA,
            pltpu.SemaphoreType.DMA,
            pltpu.SemaphoreType.DMA,
            pltpu.SemaphoreType.DMA,
        ],
    )(_gather_sum_body)
    return f(Pa, Pj, Pi, Pk, j_idx, i_idx, k_idx)


NBKT = E // 256
_NBC = 640
_CAP = 14528
_CHG = _CHUNK // 16

_UW = BOND // 2


def _bucketize_body(k_hbm, perm_hbm, kperm_hbm, cnt_hbm, loff_hbm,
                    kb_v, perm_v, kp_v, io_v, cnt_s, sem0):
    wid = lax.axis_index("s") * _NC + lax.axis_index("c")
    base = wid * _CHUNK
    lanes = lax.iota(jnp.int32, 16)
    mask0 = lanes == 0
    zero16 = jnp.zeros((16,), jnp.int32)

    def z_body(g, c):
        perm_v[pl.ds(pl.multiple_of(g * 16, 8), 16)] = zero16
        return c
    lax.fori_loop(0, _CAP // 16, z_body, 0)

    def zc_body(b, c):
        cnt_s[b] = 0
        return c
    lax.fori_loop(0, _NBC, zc_body, 0)

    def h_outer(blk5, c5):
        pltpu.async_copy(k_hbm.at[pl.ds(base + blk5 * 2000, 2000)], kb_v,
                         sem0).wait()

        def h_body(g, c):
            kv = kb_v[pl.ds(pl.multiple_of(g * 16, 8), 16)]
            for lane in range(16):
                b = kv[lane] >> 8
                cnt_s[b] = cnt_s[b] + 1
            return c
        lax.fori_loop(0, 125, h_body, 0)
        return c5
    lax.fori_loop(0, 5, h_outer, 0)

    def x1_body(b, c):
        plsc.store_scatter(io_v, [jnp.full((16,), b, jnp.int32)],
                           jnp.full((16,), cnt_s[b], jnp.int32), mask=mask0)
        return c
    lax.fori_loop(0, _NBC, x1_body, 0)
    pltpu.sync_copy(io_v, cnt_hbm.at[pl.ds(wid * _NBC, _NBC)])

    def p_body(b, run):
        c = cnt_s[b]
        cnt_s[b] = run
        return run + ((c + 7) & -8)
    lax.fori_loop(0, _NBC, p_body, jnp.int32(0))

    def x2_body(b, c):
        plsc.store_scatter(io_v, [jnp.full((16,), b, jnp.int32)],
                           jnp.full((16,), cnt_s[b], jnp.int32), mask=mask0)
        return c
    lax.fori_loop(0, _NBC, x2_body, 0)
    pltpu.sync_copy(io_v, loff_hbm.at[pl.ds(wid * _NBC, _NBC)])

    def s_outer(blk5, c5):
        pltpu.async_copy(k_hbm.at[pl.ds(base + blk5 * 2000, 2000)], kb_v,
                         sem0).wait()

        def s_body(g, c):
            g16 = pl.multiple_of(g * 16, 8)
            kv = kb_v[pl.ds(g16, 16)]
            for lane in range(16):
                k = kv[lane]
                b = k >> 8
                o = cnt_s[b]
                cnt_s[b] = o + 1
                t = base + blk5 * 2000 + g16 + lane
                oi = jnp.full((16,), o, jnp.int32)
                plsc.store_scatter(perm_v, [oi], jnp.full((16,), t, jnp.int32),
                                   mask=mask0)
                plsc.store_scatter(kp_v, [oi], jnp.full((16,), k, jnp.int32),
                                   mask=mask0)
            return c
        lax.fori_loop(0, 125, s_body, 0)
        return c5
    lax.fori_loop(0, 5, s_outer, 0)

    pltpu.sync_copy(perm_v, perm_hbm.at[pl.ds(wid * _CAP, _CAP)])
    pltpu.sync_copy(kp_v, kperm_hbm.at[pl.ds(wid * _CAP, _CAP)])


def _bucketize(k_idx):
    mesh = plsc.VectorSubcoreMesh(core_axis_name="c", subcore_axis_name="s")
    f = functools.partial(
        pl.kernel,
        mesh=mesh,
        compiler_params=pltpu.CompilerParams(needs_layout_passes=False),
        out_type=(
            jax.ShapeDtypeStruct((NW * _CAP,), jnp.int32),
            jax.ShapeDtypeStruct((NW * _CAP,), jnp.int32),
            jax.ShapeDtypeStruct((NW * _NBC,), jnp.int32),
            jax.ShapeDtypeStruct((NW * _NBC,), jnp.int32),
        ),
        scratch_types=[
            pltpu.VMEM((2000,), jnp.int32),
            pltpu.VMEM((_CAP,), jnp.int32),
            pltpu.VMEM((_CAP,), jnp.int32),
            pltpu.VMEM((_NBC,), jnp.int32),
            pltpu.SMEM((_NBC,), jnp.int32),
            pltpu.SemaphoreType.DMA,
        ],
    )(_bucketize_body)
    return f(k_idx)


def _accum_body(u_hbm, perm_hbm, kperm_hbm, cntT_hbm, loffT_hbm, seg_hbm,
                cnt_v, loff_v, acc, tbm, kbm, comb_t, comb_k, us, sem0, sem1):
    wid = lax.axis_index("s") * _NC + lax.axis_index("c")
    zero16f = jnp.zeros((16,), jnp.float32)
    zero16i = jnp.zeros((16,), jnp.int32)
    iota16 = lax.iota(jnp.int32, 16)
    brow = wid * 20
    skew = brow & 4
    start8 = pl.multiple_of(brow - skew, 8)
    c0 = pltpu.async_copy(cntT_hbm.at[pl.ds(start8, 24)], cnt_v, sem0)
    c1 = pltpu.async_copy(loffT_hbm.at[pl.ds(start8, 24)], loff_v, sem1)
    c0.wait()
    c1.wait()
    nb = jnp.maximum(0, jnp.minimum(20, NBKT - brow))

    def zc_body(g, c):
        comb_t[pl.ds(pl.multiple_of(g * 16, 8), 16)] = zero16i
        return c
    lax.fori_loop(0, 2048 // 16, zc_body, 0)

    def bkt_body(lb0, cb):
        lb = lb0 + skew
        b = brow + lb0

        def za_body(r, c):
            for c4 in range(8):
                acc[r, pl.ds(c4 * 16, 16)] = zero16f
            return c
        lax.fori_loop(0, 256, za_body, 0)

        nv0 = cnt_v[lb, pl.ds(0, 16)]
        nv1 = cnt_v[lb, pl.ds(16, 16)]
        lv0 = loff_v[lb, pl.ds(0, 16)]
        lv1 = loff_v[lb, pl.ds(16, 16)]
        total = jnp.sum(nv0) + jnp.sum(nv1)
        maxn = jnp.maximum(jnp.max(nv0), jnp.max(nv1))
        fast = jnp.logical_and(total <= 2048, maxn <= 64)

        @pl.when(jnp.logical_and(fast, total > 0))
        def _():
            hs = []
            for p in range(32):
                l = lv0[p] if p < 16 else lv1[p - 16]
                off = pl.multiple_of(p * _CAP + l, 8)
                hs.append(pltpu.async_copy(
                    perm_hbm.at[pl.ds(off, 64)], tbm.at[p], sem0))
            for h in hs:
                h.wait()
            hs = []
            for p in range(32):
                l = lv0[p] if p < 16 else lv1[p - 16]
                off = pl.multiple_of(p * _CAP + l, 8)
                hs.append(pltpu.async_copy(
                    kperm_hbm.at[pl.ds(off, 64)], kbm.at[p], sem1))
            for h in hs:
                h.wait()

            pos = jnp.int32(0)
            for p in range(32):
                n_p = nv0[p] if p < 16 else nv1[p - 16]

                def cp_body(c, pos_c, _p=p, _n=n_p):
                    vt = tbm[_p, pl.ds(c * 16, 16)]
                    vk = kbm[_p, pl.ds(c * 16, 16)]
                    msk = iota16 < (_n - c * 16)
                    plsc.store_scatter(comb_t, [pos_c + iota16], vt, mask=msk)
                    plsc.store_scatter(comb_k, [pos_c + iota16], vk, mask=msk)
                    return pos_c + jnp.minimum(16, _n - c * 16)
                pos = lax.fori_loop(0, (n_p + 15) >> 4, cp_body, pos)

            nch = (total + 127) >> 7

            def gch(ch, cc):
                pltpu.async_copy(
                    u_hbm.at[comb_t.at[pl.ds(pl.multiple_of(ch * 128, 8),
                                             128)]], us, sem0).wait()
                rem = total - ch * 128
                for g in range(8):
                    @pl.when(g * 16 < rem)
                    def _():
                        kv = comb_k[pl.ds(pl.multiple_of(ch * 128 + g * 16, 8), 16)]
                        for lane in range(16):
                            @pl.when(g * 16 + lane < rem)
                            def _():
                                r = kv[lane] & 255
                                row = g * 16 + lane
                                for c4 in range(8):
                                    sl = pl.ds(c4 * 16, 16)
                                    acc[r, sl] = acc[r, sl] + us[row, sl]
                return cc
            lax.fori_loop(0, nch, gch, 0)

        @pl.when(jnp.logical_and(jnp.logical_not(fast), total > 0))
        def _():
            def p_body(p, cp):
                n = cnt_v[lb, pl.ds(p, 16)][0]
                l = loff_v[lb, pl.ds(p, 16)][0]
                pbase = p * _CAP + l

                @pl.when(n > 0)
                def _():
                    nchunks = (n + 63) >> 6

                    def ch_body(ch, cc):
                        off = pl.multiple_of(pbase + ch * 64, 8)
                        pltpu.sync_copy(perm_hbm.at[pl.ds(off, 64)], tbm.at[0])
                        pltpu.sync_copy(kperm_hbm.at[pl.ds(off, 64)], kbm.at[0])
                        pltpu.async_copy(
                            u_hbm.at[tbm.at[0]],
                            us.at[pl.ds(0, 64)], sem0).wait()
                        rem = n - ch * 64
                        for g in range(4):
                            @pl.when(g * 16 < rem)
                            def _():
                                kv = kbm[0, pl.ds(g * 16, 16)]
                                for lane in range(16):
                                    @pl.when(g * 16 + lane < rem)
                                    def _():
                                        r = kv[lane] & 255
                                        row = g * 16 + lane
                                        for c4 in range(8):
                                            sl = pl.ds(c4 * 16, 16)
                                            acc[r, sl] = acc[r, sl] + us[row, sl]
                        return cc
                    lax.fori_loop(0, nchunks, ch_body, 0)
                return cp
            lax.fori_loop(0, NW, p_body, 0)

        pltpu.sync_copy(acc, seg_hbm.at[pl.ds(b * 256, 256)])
        return cb
    lax.fori_loop(0, nb, bkt_body, 0)


def _accumulate(u_packed, perm, kperm, cntT, loffT):
    mesh = plsc.VectorSubcoreMesh(core_axis_name="c", subcore_axis_name="s")
    f = functools.partial(
        pl.kernel,
        mesh=mesh,
        compiler_params=pltpu.CompilerParams(needs_layout_passes=False),
        out_type=jax.ShapeDtypeStruct((E, BOND), jnp.float32),
        scratch_types=[
            pltpu.VMEM((24, 48), jnp.int32),
            pltpu.VMEM((24, 48), jnp.int32),
            pltpu.VMEM((256, BOND), jnp.float32),
            pltpu.VMEM((32, 64), jnp.int32),
            pltpu.VMEM((32, 64), jnp.int32),
            pltpu.VMEM((2048,), jnp.int32),
            pltpu.VMEM((2048,), jnp.int32),
            pltpu.VMEM((128, BOND), jnp.float32),
            pltpu.SemaphoreType.DMA,
            pltpu.SemaphoreType.DMA,
        ],
    )(_accum_body)
    return f(u_packed, perm, kperm, cntT, loffT)


def _pack_pair(core_f32, gate_f32):
    cb = jax.lax.bitcast_convert_type(core_f32, jnp.uint32)
    gb = jax.lax.bitcast_convert_type(gate_f32, jnp.uint32)
    cb = (cb + jnp.uint32(0x8000)) >> 16
    gb = (gb + jnp.uint32(0x8000)) & jnp.uint32(0xFFFF0000)
    return jax.lax.bitcast_convert_type(cb | gb, jnp.int32)


def _unpack_pair(word_i32):
    w = jax.lax.bitcast_convert_type(word_i32, jnp.uint32)
    core = jax.lax.bitcast_convert_type(w << 16, jnp.float32)
    gate = jax.lax.bitcast_convert_type(w & jnp.uint32(0xFFFF0000), jnp.float32)
    return core, gate


def _mm_body(x_ref, w_ref, o_ref):
    y = jnp.dot(x_ref[...], w_ref[...], preferred_element_type=jnp.float32)
    o_ref[...] = _pack_pair(y[:, :BOND], y[:, BOND:])


def _rowmm_packed(x, w, blk):
    m, k = x.shape
    n = w.shape[1]
    return pl.pallas_call(
        _mm_body,
        grid=(m // blk,),
        in_specs=[pl.BlockSpec((blk, k), lambda i: (i, 0)),
                  pl.BlockSpec((k, n), lambda i: (0, 0))],
        out_specs=pl.BlockSpec((blk, n // 2), lambda i: (i, 0)),
        out_shape=jax.ShapeDtypeStruct((m, n // 2), jnp.int32),
    )(x, w)


def _stats_body(x_ref, o_ref):
    @pl.when(pl.program_id(0) == 0)
    def _():
        o_ref[...] = jnp.zeros_like(o_ref)
    core, gate = _unpack_pair(x_ref[...])
    o_ref[0, :] += jnp.sum(core, axis=0)
    o_ref[1, :] += jnp.sum(core * core, axis=0)
    o_ref[2, :] += jnp.sum(gate, axis=0)
    o_ref[3, :] += jnp.sum(gate * gate, axis=0)


def _stats(x, blk):
    m, n = x.shape
    return pl.pallas_call(
        _stats_body,
        grid=(m // blk,),
        in_specs=[pl.BlockSpec((blk, n), lambda i: (i, 0))],
        out_specs=pl.BlockSpec((8, n), lambda i: (0, 0)),
        out_shape=jax.ShapeDtypeStruct((8, n), jnp.float32),
    )(x)


def _act_body(x_ref, st_ref, p_ref, o_ref):
    core, gate = _unpack_pair(x_ref[...])
    mean_c = st_ref[0, :] / T
    var_c = st_ref[1, :] / T - mean_c * mean_c
    inv_c = jax.lax.rsqrt(var_c + 1e-5)
    mean_g = st_ref[2, :] / T
    var_g = st_ref[3, :] / T - mean_g * mean_g
    inv_g = jax.lax.rsqrt(var_g + 1e-5)
    core = (core - mean_c) * inv_c * p_ref[0, :] + p_ref[1, :]
    gate = (gate - mean_g) * inv_g * p_ref[2, :] + p_ref[3, :]
    core = core * jax.nn.sigmoid(core)
    gate = jax.nn.sigmoid(gate)
    o_ref[...] = core * gate


def _activate(x, stats, params, blk):
    m = x.shape[0]
    return pl.pallas_call(
        _act_body,
        grid=(m // blk,),
        in_specs=[pl.BlockSpec((blk, BOND), lambda i: (i, 0)),
                  pl.BlockSpec((8, BOND), lambda i: (0, 0)),
                  pl.BlockSpec((8, BOND), lambda i: (0, 0))],
        out_specs=pl.BlockSpec((blk, BOND), lambda i: (i, 0)),
        out_shape=jax.ShapeDtypeStruct((m, BOND), jnp.float32),
    )(x, stats, params)


def _final_body(seg_ref, edge_ref, w_ref, out_ref):
    out_ref[...] = jnp.dot(seg_ref[...], w_ref[...],
                           preferred_element_type=jnp.float32) + edge_ref[...]


def _final_matmul(segsum, edge_feat, W_out):
    BLK = 1600
    return pl.pallas_call(
        _final_body,
        grid=(E // BLK,),
        in_specs=[
            pl.BlockSpec((BLK, BOND), lambda i: (i, 0)),
            pl.BlockSpec((BLK, BOND), lambda i: (i, 0)),
            pl.BlockSpec((BOND, BOND), lambda i: (0, 0)),
        ],
        out_specs=pl.BlockSpec((BLK, BOND), lambda i: (i, 0)),
        out_shape=jax.ShapeDtypeStruct((E, BOND), jnp.float32),
    )(segsum, edge_feat, W_out)


def kernel(vertex_feat, edge_feat, angle_feat, edge_index, k_idx, j_idx, i_idx,
           W_core_src, W_core_dst, W_core_bond, W_core_angle,
           W_gate_src, W_gate_dst, W_gate_bond, W_gate_angle,
           bn_core_gamma, bn_core_beta, bn_gate_gamma, bn_gate_beta, W_out):
    k_idx = k_idx.astype(jnp.int32)
    j_idx = j_idx.astype(jnp.int32)
    i_idx = i_idx.astype(jnp.int32)

    Wj = jnp.concatenate([W_core_src, W_gate_src], axis=1)
    Wi = jnp.concatenate([W_core_dst, W_gate_dst], axis=1)
    Wk = jnp.concatenate([W_core_bond, W_gate_bond], axis=1)
    Wa = jnp.concatenate([W_core_angle, W_gate_angle], axis=1)
    Pj = _rowmm_packed(vertex_feat, Wj, 2000)
    Pi = _rowmm_packed(vertex_feat, Wi, 2000)
    Pk = _rowmm_packed(edge_feat, Wk, 4000)
    Pa = _rowmm_packed(angle_feat, Wa, 8000)

    x = _gather_sum(Pa, Pj, Pi, Pk, j_idx, i_idx, k_idx)

    stats = _stats(x, 8000)
    params = jnp.zeros((8, BOND), jnp.float32)
    params = params.at[0].set(bn_core_gamma).at[1].set(bn_core_beta)
    params = params.at[2].set(bn_gate_gamma).at[3].set(bn_gate_beta)
    u_packed = _activate(x, stats, params, 4000)

    perm, kperm, cnt, loff = _bucketize(k_idx)
    pad = jnp.zeros((_NBC, 48 - NW), jnp.int32)
    cntT = jnp.concatenate([cnt.reshape(NW, _NBC).T, pad], axis=1)
    loffT = jnp.concatenate([loff.reshape(NW, _NBC).T, pad], axis=1)
    segsum = _accumulate(u_packed, perm, kperm, cntT, loffT)

    return _final_matmul(segsum, edge_feat, W_out)


# speedup vs baseline: 3.5367x; 1.0838x over previous
import functools

import jax
import jax.numpy as jnp
from jax import lax
from jax.experimental import pallas as pl
from jax.experimental.pallas import tpu as pltpu
from jax.experimental.pallas import tpu_sc as plsc

N, E, T = 10000, 160000, 320000
ATOM, BOND, ANGLE = 128, 128, 16
C2 = 2 * BOND

_SC_INFO = plsc.get_sparse_core_info()
_NC = _SC_INFO.num_cores
_NS = _SC_INFO.num_subcores
NW = _NC * _NS


_GB = 80
_CHUNK = T // NW
_NBLK = _CHUNK // _GB


_CW = C2 // 2


def _bf2(v):
    return plsc.bitcast(v, jnp.bfloat16)


def _gather_sum_body(pa_hbm, pj_hbm, pi_hbm, pk_hbm, j_hbm, i_hbm, k_hbm,
                     x_hbm, jb, ib, kb, xa, gj, gi, gk,
                     sem_a, sem_j, sem_i, sem_k):
    wid = lax.axis_index("s") * _NC + lax.axis_index("c")

    def blk_body(b, carry):
        base = wid * _CHUNK + b * _GB
        pltpu.sync_copy(j_hbm.at[pl.ds(base, _GB)], jb)
        pltpu.sync_copy(i_hbm.at[pl.ds(base, _GB)], ib)
        pltpu.sync_copy(k_hbm.at[pl.ds(base, _GB)], kb)
        ca = pltpu.async_copy(pa_hbm.at[pl.ds(base, _GB)], xa, sem_a)
        cj = pltpu.async_copy(pj_hbm.at[jb], gj, sem_j)
        ci = pltpu.async_copy(pi_hbm.at[ib], gi, sem_i)
        ck = pltpu.async_copy(pk_hbm.at[kb], gk, sem_k)
        ca.wait()
        cj.wait()
        ci.wait()
        ck.wait()

        def row_body(r, c2):
            for c in range(_CW // 16):
                sl = pl.ds(c * 16, 16)
                s = (_bf2(xa[r, sl]) + _bf2(gj[r, sl])) +                     (_bf2(gi[r, sl]) + _bf2(gk[r, sl]))
                xa[r, sl] = plsc.bitcast(s, jnp.int32)
            return c2

        lax.fori_loop(0, _GB, row_body, 0)
        pltpu.sync_copy(xa, x_hbm.at[pl.ds(base, _GB)])
        return carry

    lax.fori_loop(0, _NBLK, blk_body, 0)


def _gather_sum(Pa, Pj, Pi, Pk, j_idx, i_idx, k_idx):
    mesh = plsc.VectorSubcoreMesh(core_axis_name="c", subcore_axis_name="s")
    f = functools.partial(
        pl.kernel,
        mesh=mesh,
        compiler_params=pltpu.CompilerParams(needs_layout_passes=False),
        out_type=jax.ShapeDtypeStruct((T, _CW), jnp.int32),
        scratch_types=[
            pltpu.VMEM((_GB,), jnp.int32),
            pltpu.VMEM((_GB,), jnp.int32),
            pltpu.VMEM((_GB,), jnp.int32),
            pltpu.VMEM((_GB, _CW), jnp.int32),
            pltpu.VMEM((_GB, _CW), jnp.int32),
            pltpu.VMEM((_GB, _CW), jnp.int32),
            pltpu.VMEM((_GB, _CW), jnp.int32),
            pltpu.SemaphoreType.DMA,
            pltpu.SemaphoreType.DMA,
            pltpu.SemaphoreType.DMA,
            pltpu.SemaphoreType.DMA,
        ],
    )(_gather_sum_body)
    return f(Pa, Pj, Pi, Pk, j_idx, i_idx, k_idx)


NBKT = E // 256
_NBC = 640
_CAP = 14528
_CHG = _CHUNK // 16

_UW = BOND // 2


def _bucketize_body(k_hbm, perm_hbm, kperm_hbm, cnt_hbm, loff_hbm,
                    kb_v, perm_v, kp_v, cnt_s, sem0):
    wid = lax.axis_index("s") * _NC + lax.axis_index("c")
    base = wid * _CHUNK
    lanes = lax.iota(jnp.int32, 16)
    mask0 = lanes == 0
    zero16 = jnp.zeros((16,), jnp.int32)

    def z_body(g, c):
        perm_v[pl.ds(pl.multiple_of(g * 16, 8), 16)] = zero16
        return c
    lax.fori_loop(0, _CAP // 16, z_body, 0)

    def zc_body(b, c):
        cnt_s[b] = 0
        return c
    lax.fori_loop(0, _NBC, zc_body, 0)

    def h_outer(blk5, c5):
        pltpu.async_copy(k_hbm.at[pl.ds(base + blk5 * 2000, 2000)], kb_v,
                         sem0).wait()

        def h_body(g, c):
            kv = kb_v[pl.ds(pl.multiple_of(g * 16, 8), 16)]
            for lane in range(16):
                b = kv[lane] >> 8
                cnt_s[b] = cnt_s[b] + 1
            return c
        lax.fori_loop(0, 125, h_body, 0)
        return c5
    lax.fori_loop(0, 5, h_outer, 0)

    def x1_body(b, c):
        plsc.store_scatter(kb_v, [jnp.full((16,), b, jnp.int32)],
                           jnp.full((16,), cnt_s[b], jnp.int32), mask=mask0)
        return c
    lax.fori_loop(0, _NBC, x1_body, 0)
    pltpu.sync_copy(kb_v.at[pl.ds(0, _NBC)], cnt_hbm.at[pl.ds(wid * _NBC, _NBC)])

    def p_body(b, run):
        c = cnt_s[b]
        cnt_s[b] = run
        return run + ((c + 7) & -8)
    lax.fori_loop(0, _NBC, p_body, jnp.int32(0))

    def x2_body(b, c):
        plsc.store_scatter(kb_v, [jnp.full((16,), b, jnp.int32)],
                           jnp.full((16,), cnt_s[b], jnp.int32), mask=mask0)
        return c
    lax.fori_loop(0, _NBC, x2_body, 0)
    pltpu.sync_copy(kb_v.at[pl.ds(0, _NBC)], loff_hbm.at[pl.ds(wid * _NBC, _NBC)])

    def s_outer(blk5, c5):
        pltpu.async_copy(k_hbm.at[pl.ds(base + blk5 * 2000, 2000)], kb_v,
                         sem0).wait()

        def s_body(g, c):
            g16 = pl.multiple_of(g * 16, 8)
            kv = kb_v[pl.ds(g16, 16)]
            for lane in range(16):
                k = kv[lane]
                b = k >> 8
                o = cnt_s[b]
                cnt_s[b] = o + 1
                t = base + blk5 * 2000 + g16 + lane
                oi = jnp.full((16,), o, jnp.int32)
                plsc.store_scatter(perm_v, [oi], jnp.full((16,), t, jnp.int32),
                                   mask=mask0)
                plsc.store_scatter(kp_v, [oi], jnp.full((16,), k, jnp.int32),
                                   mask=mask0)
            return c
        lax.fori_loop(0, 125, s_body, 0)
        return c5
    lax.fori_loop(0, 5, s_outer, 0)

    pltpu.sync_copy(perm_v, perm_hbm.at[pl.ds(wid * _CAP, _CAP)])
    pltpu.sync_copy(kp_v, kperm_hbm.at[pl.ds(wid * _CAP, _CAP)])


def _bucketize(k_idx):
    mesh = plsc.VectorSubcoreMesh(core_axis_name="c", subcore_axis_name="s")
    f = functools.partial(
        pl.kernel,
        mesh=mesh,
        compiler_params=pltpu.CompilerParams(needs_layout_passes=False),
        out_type=(
            jax.ShapeDtypeStruct((NW * _CAP,), jnp.int32),
            jax.ShapeDtypeStruct((NW * _CAP,), jnp.int32),
            jax.ShapeDtypeStruct((NW * _NBC,), jnp.int32),
            jax.ShapeDtypeStruct((NW * _NBC,), jnp.int32),
        ),
        scratch_types=[
            pltpu.VMEM((2000,), jnp.int32),
            pltpu.VMEM((_CAP,), jnp.int32),
            pltpu.VMEM((_CAP,), jnp.int32),
            pltpu.SMEM((_NBC,), jnp.int32),
            pltpu.SemaphoreType.DMA,
        ],
    )(_bucketize_body)
    return f(k_idx)


def _accum_body(u_hbm, perm_hbm, kperm_hbm, cntT_hbm, loffT_hbm, seg_hbm,
                cnt_v, loff_v, acc, tbm, kbm, comb_t, comb_k, us, sem0, sem1):
    wid = lax.axis_index("s") * _NC + lax.axis_index("c")
    zero16f = jnp.zeros((16,), jnp.float32)
    zero16i = jnp.zeros((16,), jnp.int32)
    iota16 = lax.iota(jnp.int32, 16)
    brow = wid * 20
    skew = brow & 4
    start8 = pl.multiple_of(brow - skew, 8)
    c0 = pltpu.async_copy(cntT_hbm.at[pl.ds(start8, 24)], cnt_v, sem0)
    c1 = pltpu.async_copy(loffT_hbm.at[pl.ds(start8, 24)], loff_v, sem1)
    c0.wait()
    c1.wait()
    nb = jnp.maximum(0, jnp.minimum(20, NBKT - brow))

    def zc_body(g, c):
        comb_t[pl.ds(pl.multiple_of(g * 16, 8), 16)] = zero16i
        return c
    lax.fori_loop(0, 1280 // 16, zc_body, 0)

    def bkt_body(lb0, cb):
        lb = lb0 + skew
        b = brow + lb0

        def za_body(r, c):
            for c4 in range(8):
                acc[r, pl.ds(c4 * 16, 16)] = zero16f
            return c
        lax.fori_loop(0, 256, za_body, 0)

        nv0 = cnt_v[lb, pl.ds(0, 16)]
        nv1 = cnt_v[lb, pl.ds(16, 16)]
        lv0 = loff_v[lb, pl.ds(0, 16)]
        lv1 = loff_v[lb, pl.ds(16, 16)]
        total = jnp.sum(nv0) + jnp.sum(nv1)
        maxn = jnp.maximum(jnp.max(nv0), jnp.max(nv1))
        fast = jnp.logical_and(total <= 1280, maxn <= 64)

        @pl.when(jnp.logical_and(fast, total > 0))
        def _():
            hs = []
            for p in range(32):
                l = lv0[p] if p < 16 else lv1[p - 16]
                off = pl.multiple_of(p * _CAP + l, 8)
                hs.append(pltpu.async_copy(
                    perm_hbm.at[pl.ds(off, 64)], tbm.at[p], sem0))
            for h in hs:
                h.wait()
            hs = []
            for p in range(32):
                l = lv0[p] if p < 16 else lv1[p - 16]
                off = pl.multiple_of(p * _CAP + l, 8)
                hs.append(pltpu.async_copy(
                    kperm_hbm.at[pl.ds(off, 64)], kbm.at[p], sem1))
            for h in hs:
                h.wait()

            pos = jnp.int32(0)
            for p in range(32):
                n_p = nv0[p] if p < 16 else nv1[p - 16]

                def cp_body(c, pos_c, _p=p, _n=n_p):
                    vt = tbm[_p, pl.ds(c * 16, 16)]
                    vk = kbm[_p, pl.ds(c * 16, 16)]
                    msk = iota16 < (_n - c * 16)
                    plsc.store_scatter(comb_t, [pos_c + iota16], vt, mask=msk)
                    plsc.store_scatter(comb_k, [pos_c + iota16], vk, mask=msk)
                    return pos_c + jnp.minimum(16, _n - c * 16)
                pos = lax.fori_loop(0, (n_p + 15) >> 4, cp_body, pos)

            nch = (total + 127) >> 7

            def gch(ch, cc):
                pltpu.async_copy(
                    u_hbm.at[comb_t.at[pl.ds(pl.multiple_of(ch * 128, 8),
                                             128)]], us, sem0).wait()
                rem = total - ch * 128
                for g in range(8):
                    @pl.when(g * 16 < rem)
                    def _():
                        kv = comb_k[pl.ds(pl.multiple_of(ch * 128 + g * 16, 8), 16)]
                        for lane in range(16):
                            @pl.when(g * 16 + lane < rem)
                            def _():
                                r = kv[lane] & 255
                                row = g * 16 + lane
                                for c4 in range(8):
                                    sl = pl.ds(c4 * 16, 16)
                                    acc[r, sl] = acc[r, sl] + us[row, sl]
                return cc
            lax.fori_loop(0, nch, gch, 0)

        @pl.when(jnp.logical_and(jnp.logical_not(fast), total > 0))
        def _():
            def p_body(p, cp):
                n = cnt_v[lb, pl.ds(p, 16)][0]
                l = loff_v[lb, pl.ds(p, 16)][0]
                pbase = p * _CAP + l

                @pl.when(n > 0)
                def _():
                    nchunks = (n + 63) >> 6

                    def ch_body(ch, cc):
                        off = pl.multiple_of(pbase + ch * 64, 8)
                        pltpu.sync_copy(perm_hbm.at[pl.ds(off, 64)], tbm.at[0])
                        pltpu.sync_copy(kperm_hbm.at[pl.ds(off, 64)], kbm.at[0])
                        pltpu.async_copy(
                            u_hbm.at[tbm.at[0]],
                            us.at[pl.ds(0, 64)], sem0).wait()
                        rem = n - ch * 64
                        for g in range(4):
                            @pl.when(g * 16 < rem)
                            def _():
                                kv = kbm[0, pl.ds(g * 16, 16)]
                                for lane in range(16):
                                    @pl.when(g * 16 + lane < rem)
                                    def _():
                                        r = kv[lane] & 255
                                        row = g * 16 + lane
                                        for c4 in range(8):
                                            sl = pl.ds(c4 * 16, 16)
                                            acc[r, sl] = acc[r, sl] + us[row, sl]
                        return cc
                    lax.fori_loop(0, nchunks, ch_body, 0)
                return cp
            lax.fori_loop(0, NW, p_body, 0)

        pltpu.sync_copy(acc, seg_hbm.at[pl.ds(b * 256, 256)])
        return cb
    lax.fori_loop(0, nb, bkt_body, 0)


def _accumulate(u_packed, perm, kperm, cntT, loffT):
    mesh = plsc.VectorSubcoreMesh(core_axis_name="c", subcore_axis_name="s")
    f = functools.partial(
        pl.kernel,
        mesh=mesh,
        compiler_params=pltpu.CompilerParams(needs_layout_passes=False),
        out_type=jax.ShapeDtypeStruct((E, BOND), jnp.float32),
        scratch_types=[
            pltpu.VMEM((24, 48), jnp.int32),
            pltpu.VMEM((24, 48), jnp.int32),
            pltpu.VMEM((256, BOND), jnp.float32),
            pltpu.VMEM((32, 64), jnp.int32),
            pltpu.VMEM((32, 64), jnp.int32),
            pltpu.VMEM((1280,), jnp.int32),
            pltpu.VMEM((1280,), jnp.int32),
            pltpu.VMEM((128, BOND), jnp.float32),
            pltpu.SemaphoreType.DMA,
            pltpu.SemaphoreType.DMA,
        ],
    )(_accum_body)
    return f(u_packed, perm, kperm, cntT, loffT)


def _pack_pair(core_f32, gate_f32):
    cb = jax.lax.bitcast_convert_type(core_f32, jnp.uint32)
    gb = jax.lax.bitcast_convert_type(gate_f32, jnp.uint32)
    cb = (cb + jnp.uint32(0x8000)) >> 16
    gb = (gb + jnp.uint32(0x8000)) & jnp.uint32(0xFFFF0000)
    return jax.lax.bitcast_convert_type(cb | gb, jnp.int32)


def _unpack_pair(word_i32):
    w = jax.lax.bitcast_convert_type(word_i32, jnp.uint32)
    core = jax.lax.bitcast_convert_type(w << 16, jnp.float32)
    gate = jax.lax.bitcast_convert_type(w & jnp.uint32(0xFFFF0000), jnp.float32)
    return core, gate


def _mm_body(x_ref, w_ref, o_ref):
    y = jnp.dot(x_ref[...], w_ref[...], preferred_element_type=jnp.float32)
    o_ref[...] = _pack_pair(y[:, :BOND], y[:, BOND:])


def _rowmm_packed(x, w, blk):
    m, k = x.shape
    n = w.shape[1]
    return pl.pallas_call(
        _mm_body,
        grid=(m // blk,),
        in_specs=[pl.BlockSpec((blk, k), lambda i: (i, 0)),
                  pl.BlockSpec((k, n), lambda i: (0, 0))],
        out_specs=pl.BlockSpec((blk, n // 2), lambda i: (i, 0)),
        out_shape=jax.ShapeDtypeStruct((m, n // 2), jnp.int32),
    )(x, w)


def _stats_body(x_ref, o_ref):
    @pl.when(pl.program_id(0) == 0)
    def _():
        o_ref[...] = jnp.zeros_like(o_ref)
    core, gate = _unpack_pair(x_ref[...])
    o_ref[0, :] += jnp.sum(core, axis=0)
    o_ref[1, :] += jnp.sum(core * core, axis=0)
    o_ref[2, :] += jnp.sum(gate, axis=0)
    o_ref[3, :] += jnp.sum(gate * gate, axis=0)


def _stats(x, blk):
    m, n = x.shape
    return pl.pallas_call(
        _stats_body,
        grid=(m // blk,),
        in_specs=[pl.BlockSpec((blk, n), lambda i: (i, 0))],
        out_specs=pl.BlockSpec((8, n), lambda i: (0, 0)),
        out_shape=jax.ShapeDtypeStruct((8, n), jnp.float32),
    )(x)


def _act_body(x_ref, st_ref, p_ref, o_ref):
    core, gate = _unpack_pair(x_ref[...])
    mean_c = st_ref[0, :] / T
    var_c = st_ref[1, :] / T - mean_c * mean_c
    inv_c = jax.lax.rsqrt(var_c + 1e-5)
    mean_g = st_ref[2, :] / T
    var_g = st_ref[3, :] / T - mean_g * mean_g
    inv_g = jax.lax.rsqrt(var_g + 1e-5)
    core = (core - mean_c) * inv_c * p_ref[0, :] + p_ref[1, :]
    gate = (gate - mean_g) * inv_g * p_ref[2, :] + p_ref[3, :]
    core = core * jax.nn.sigmoid(core)
    gate = jax.nn.sigmoid(gate)
    o_ref[...] = core * gate


def _activate(x, stats, params, blk):
    m = x.shape[0]
    return pl.pallas_call(
        _act_body,
        grid=(m // blk,),
        in_specs=[pl.BlockSpec((blk, BOND), lambda i: (i, 0)),
                  pl.BlockSpec((8, BOND), lambda i: (0, 0)),
                  pl.BlockSpec((8, BOND), lambda i: (0, 0))],
        out_specs=pl.BlockSpec((blk, BOND), lambda i: (i, 0)),
        out_shape=jax.ShapeDtypeStruct((m, BOND), jnp.float32),
    )(x, stats, params)


def _final_body(seg_ref, edge_ref, w_ref, out_ref):
    out_ref[...] = jnp.dot(seg_ref[...], w_ref[...],
                           preferred_element_type=jnp.float32) + edge_ref[...]


def _final_matmul(segsum, edge_feat, W_out):
    BLK = 1600
    return pl.pallas_call(
        _final_body,
        grid=(E // BLK,),
        in_specs=[
            pl.BlockSpec((BLK, BOND), lambda i: (i, 0)),
            pl.BlockSpec((BLK, BOND), lambda i: (i, 0)),
            pl.BlockSpec((BOND, BOND), lambda i: (0, 0)),
        ],
        out_specs=pl.BlockSpec((BLK, BOND), lambda i: (i, 0)),
        out_shape=jax.ShapeDtypeStruct((E, BOND), jnp.float32),
    )(segsum, edge_feat, W_out)


def kernel(vertex_feat, edge_feat, angle_feat, edge_index, k_idx, j_idx, i_idx,
           W_core_src, W_core_dst, W_core_bond, W_core_angle,
           W_gate_src, W_gate_dst, W_gate_bond, W_gate_angle,
           bn_core_gamma, bn_core_beta, bn_gate_gamma, bn_gate_beta, W_out):
    k_idx = k_idx.astype(jnp.int32)
    j_idx = j_idx.astype(jnp.int32)
    i_idx = i_idx.astype(jnp.int32)

    Wj = jnp.concatenate([W_core_src, W_gate_src], axis=1)
    Wi = jnp.concatenate([W_core_dst, W_gate_dst], axis=1)
    Wk = jnp.concatenate([W_core_bond, W_gate_bond], axis=1)
    Wa = jnp.concatenate([W_core_angle, W_gate_angle], axis=1)
    Pj = _rowmm_packed(vertex_feat, Wj, 2000)
    Pi = _rowmm_packed(vertex_feat, Wi, 2000)
    Pk = _rowmm_packed(edge_feat, Wk, 4000)
    Pa = _rowmm_packed(angle_feat, Wa, 8000)

    x = _gather_sum(Pa, Pj, Pi, Pk, j_idx, i_idx, k_idx)

    stats = _stats(x, 8000)
    params = jnp.zeros((8, BOND), jnp.float32)
    params = params.at[0].set(bn_core_gamma).at[1].set(bn_core_beta)
    params = params.at[2].set(bn_gate_gamma).at[3].set(bn_gate_beta)
    u_packed = _activate(x, stats, params, 4000)

    perm, kperm, cnt, loff = _bucketize(k_idx)
    pad = jnp.zeros((_NBC, 48 - NW), jnp.int32)
    cntT = jnp.concatenate([cnt.reshape(NW, _NBC).T, pad], axis=1)
    loffT = jnp.concatenate([loff.reshape(NW, _NBC).T, pad], axis=1)
    segsum = _accumulate(u_packed, perm, kperm, cntT, loffT)

    return _final_matmul(segsum, edge_feat, W_out)
